# baseline (speedup 1.0000x reference)
import functools

import numpy as np

import jax
import jax.numpy as jnp
from jax import lax
from jax.experimental import pallas as pl
from jax.experimental.pallas import tpu as pltpu
from jax.experimental.pallas import tpu_sc as plsc

_N = 50000
_E = 800000
_D_IN = 128
_ZDIM = 96
_ENC_DIM = 16
_PE_ALPHA = 0.1

_NC = 2
_NS = 16
_NW = _NC * _NS

_NP = 50048
_ZR = _NP // _NS

_EPAD = 819200
_EROWS = _EPAD // 128
_RB = 8
_NBLK = _EROWS // _RB
_T0 = 400

_BLK = 2000
_GRID = _N // _BLK


def _sc_mesh():
    return plsc.VectorSubcoreMesh(core_axis_name="c", subcore_axis_name="s")


def _worker_blocks(cid, sid):
    tc = jnp.where(cid == 0, _T0, _NBLK - _T0)
    base = jnp.where(cid == 0, 0, _T0)
    per = tc // _NS
    rem = tc - per * _NS
    start = base + sid * per + jnp.minimum(sid, rem)
    nblk = per + (sid < rem).astype(jnp.int32)
    return start, nblk


def _sc_degree_count(dst2d, ones_col, zeros_col):
    @functools.partial(
        pl.kernel,
        out_type=jax.ShapeDtypeStruct((_NC, _NP, 1), jnp.float32),
        mesh=_sc_mesh(),
        scratch_types=[
            pltpu.VMEM((_RB, 128), jnp.int32),
            pltpu.VMEM((128, 1), jnp.float32),
            pltpu.VMEM_SHARED((_NP, 1), jnp.float32),
            pltpu.SemaphoreType.DMA,
        ],
        compiler_params=pltpu.CompilerParams(use_tc_tiling_on_sc=False),
    )
    def k(dst_h, ones_h, zeros_h, out_h, idx_d, ones_v, acc, ssem):
        cid = lax.axis_index("c")
        sid = lax.axis_index("s")
        pltpu.sync_copy(zeros_h, acc.at[pl.ds(sid * _ZR, _ZR)])
        pltpu.sync_copy(ones_h, ones_v)
        plsc.subcore_barrier()
        start, nblk = _worker_blocks(cid, sid)

        def body(b, carry):
            pltpu.sync_copy(dst_h.at[pl.ds((start + b) * _RB, _RB)], idx_d)
            descs = [
                pltpu.async_copy(ones_v, acc.at[idx_d.at[j]], ssem, add=True)
                for j in range(_RB)
            ]
            for d in descs:
                d.wait()
            return carry

        lax.fori_loop(0, nblk, body, 0)
        plsc.subcore_barrier()
        pltpu.sync_copy(acc.at[pl.ds(sid * _ZR, _ZR)],
                        out_h.at[cid, pl.ds(sid * _ZR, _ZR)])

    return k(dst2d, ones_col, zeros_col)


def _sc_segment_sum(table, src2d, dst2d, zeros, width):
    @functools.partial(
        pl.kernel,
        out_type=jax.ShapeDtypeStruct((_NC, _NP, width), jnp.float32),
        mesh=_sc_mesh(),
        scratch_types=[
            pltpu.VMEM((_RB, 128), jnp.int32),
            pltpu.VMEM((_RB, 128), jnp.int32),
            pltpu.VMEM((_RB * 128, width), jnp.float32),
            pltpu.VMEM_SHARED((_NP, width), jnp.float32),
            pltpu.SemaphoreType.DMA,
            pltpu.SemaphoreType.DMA,
        ],
        compiler_params=pltpu.CompilerParams(use_tc_tiling_on_sc=False),
    )
    def k(table_h, src_h, dst_h, zeros_h, out_h,
          idx_s, idx_d, rows, acc, gsem, ssem):
        cid = lax.axis_index("c")
        sid = lax.axis_index("s")
        pltpu.sync_copy(zeros_h, acc.at[pl.ds(sid * _ZR, _ZR)])
        plsc.subcore_barrier()
        start, nblk = _worker_blocks(cid, sid)

        def body(b, carry):
            rbase = (start + b) * _RB
            pltpu.sync_copy(src_h.at[pl.ds(rbase, _RB)], idx_s)
            pltpu.sync_copy(dst_h.at[pl.ds(rbase, _RB)], idx_d)
            gd = [
                pltpu.async_copy(table_h.at[idx_s.at[j]],
                                 rows.at[pl.ds(j * 128, 128)], gsem)
                for j in range(_RB)
            ]
            for d in gd:
                d.wait()
            sd = [
                pltpu.async_copy(rows.at[pl.ds(j * 128, 128)],
                                 acc.at[idx_d.at[j]], ssem, add=True)
                for j in range(_RB)
            ]
            for d in sd:
                d.wait()
            return carry

        lax.fori_loop(0, nblk, body, 0)
        plsc.subcore_barrier()
        pltpu.sync_copy(acc.at[pl.ds(sid * _ZR, _ZR)],
                        out_h.at[cid, pl.ds(sid * _ZR, _ZR)])

    return k(table, src2d, dst2d, zeros)


def _tc_encoder(x, coords, cnt, W1, b1r, W2, b2r, Wg1, fvec, smask):
    def body(x_r, c_r, cnt0_r, cnt1_r, W1_r, b1_r, W2_r, b2_r, Wg1_r,
             fv_r, sm_r, po_r, hs1a_r, hs1b_r, di_r):
        xb = x_r[...]
        e0 = jnp.maximum(
            jnp.dot(xb, W1_r[...], preferred_element_type=jnp.float32)
            + b1_r[...], 0.0)
        fe = jnp.maximum(
            jnp.dot(e0, W2_r[...], preferred_element_type=jnp.float32)
            + b2_r[...], 0.0)
        g = jnp.dot(fe, Wg1_r[...], preferred_element_type=jnp.float32)
        deg = cnt0_r[0] + cnt1_r[0] + 1.0
        di = lax.rsqrt(deg)
        di_r[...] = di
        hs1 = g * di
        hs1a_r[...] = hs1[:, 0:16]
        hs1b_r[...] = hs1[:, 16:32]
        c = c_r[...]
        crep = jnp.concatenate(
            [jnp.broadcast_to(c[:, k:k + 1], (_BLK, 32)) for k in range(3)],
            axis=1)
        ang = crep * fv_r[...]
        sm = sm_r[...]
        po_r[...] = sm * jnp.sin(ang) + (1.0 - sm) * jnp.cos(ang) + fe

    return pl.pallas_call(
        body,
        grid=(_GRID,),
        in_specs=[
            pl.BlockSpec((_BLK, _D_IN), lambda i: (i, 0)),
            pl.BlockSpec((_BLK, 3), lambda i: (i, 0)),
            pl.BlockSpec((1, _BLK, 1), lambda i: (0, i, 0)),
            pl.BlockSpec((1, _BLK, 1), lambda i: (1, i, 0)),
            pl.BlockSpec((_D_IN, 512), lambda i: (0, 0)),
            pl.BlockSpec((1, 512), lambda i: (0, 0)),
            pl.BlockSpec((512, _ZDIM), lambda i: (0, 0)),
            pl.BlockSpec((1, _ZDIM), lambda i: (0, 0)),
            pl.BlockSpec((_ZDIM, 32), lambda i: (0, 0)),
            pl.BlockSpec((1, _ZDIM), lambda i: (0, 0)),
            pl.BlockSpec((1, _ZDIM), lambda i: (0, 0)),
        ],
        out_specs=[
            pl.BlockSpec((_BLK, _ZDIM), lambda i: (i, 0)),
            pl.BlockSpec((_BLK, 16), lambda i: (i, 0)),
            pl.BlockSpec((_BLK, 16), lambda i: (i, 0)),
            pl.BlockSpec((_BLK, 1), lambda i: (i, 0)),
        ],
        out_shape=[
            jax.ShapeDtypeStruct((_N, _ZDIM), jnp.float32),
            jax.ShapeDtypeStruct((_N, 16), jnp.float32),
            jax.ShapeDtypeStruct((_N, 16), jnp.float32),
            jax.ShapeDtypeStruct((_N, 1), jnp.float32),
        ],
    )(x, coords, cnt, cnt, W1, b1r, W2, b2r, Wg1, fvec, smask)


def _tc_hidden(s1a, s1b, hs1a, hs1b, dinv, Wcat):
    def body(s1a0_r, s1a1_r, s1b0_r, s1b1_r, hs1a_r, hs1b_r, di_r, Wc_r,
             hs2_r):
        di = di_r[...]
        ha = di * (s1a0_r[0] + s1a1_r[0] + hs1a_r[...])
        hb = di * (s1b0_r[0] + s1b1_r[0] + hs1b_r[...])
        h = jnp.tanh(jnp.concatenate([ha, hb], axis=1))
        hs2_r[...] = jnp.dot(h, Wc_r[...],
                             preferred_element_type=jnp.float32) * di

    half = pl.BlockSpec((_BLK, 16), lambda i: (i, 0))
    return pl.pallas_call(
        body,
        grid=(_GRID,),
        in_specs=[
            pl.BlockSpec((1, _BLK, 16), lambda i: (0, i, 0)),
            pl.BlockSpec((1, _BLK, 16), lambda i: (1, i, 0)),
            pl.BlockSpec((1, _BLK, 16), lambda i: (0, i, 0)),
            pl.BlockSpec((1, _BLK, 16), lambda i: (1, i, 0)),
            half,
            half,
            pl.BlockSpec((_BLK, 1), lambda i: (i, 0)),
            pl.BlockSpec((32, 16), lambda i: (0, 0)),
        ],
        out_specs=[pl.BlockSpec((_BLK, 16), lambda i: (i, 0))],
        out_shape=[jax.ShapeDtypeStruct((_N, 16), jnp.float32)],
    )(s1a, s1a, s1b, s1b, hs1a, hs1b, dinv, Wcat)[0]


def _tc_final(s2, hs2, dinv):
    def body(s20_r, s21_r, hs2_r, di_r, o_r):
        o_r[...] = di_r[...] * (s20_r[0] + s21_r[0] + hs2_r[...])

    return pl.pallas_call(
        body,
        grid=(_GRID,),
        in_specs=[
            pl.BlockSpec((1, _BLK, 16), lambda i: (0, i, 0)),
            pl.BlockSpec((1, _BLK, 16), lambda i: (1, i, 0)),
            pl.BlockSpec((_BLK, 16), lambda i: (i, 0)),
            pl.BlockSpec((_BLK, 1), lambda i: (i, 0)),
        ],
        out_specs=[pl.BlockSpec((_BLK, 16), lambda i: (i, 0))],
        out_shape=[jax.ShapeDtypeStruct((_N, 16), jnp.float32)],
    )(s2, s2, hs2, dinv)[0]


def kernel(x, edge_index, coords, W1, b1, W2, b2, Wg1, Wg2, Wg3):
    pad = _EPAD - _E
    src2d = jnp.concatenate(
        [edge_index[0], jnp.zeros((pad,), jnp.int32)]).reshape(_EROWS, 128)
    dst2d = jnp.concatenate(
        [edge_index[1], jnp.full((pad,), _N, jnp.int32)]).reshape(_EROWS, 128)

    zeros16 = jnp.zeros((_ZR, 16), jnp.float32)
    zeros1 = jnp.zeros((_ZR, 1), jnp.float32)
    ones_col = jnp.ones((128, 1), jnp.float32)

    kk = np.arange(_ZDIM) % 32
    fvec = jnp.asarray(
        (_PE_ALPHA * (2.0 ** (kk % _ENC_DIM))).reshape(1, _ZDIM), jnp.float32)
    smask = jnp.asarray((kk < _ENC_DIM).astype(np.float32).reshape(1, _ZDIM))

    b1r = b1.reshape(1, 512)
    b2r = b2.reshape(1, _ZDIM)
    Wcat = jnp.pad(jnp.concatenate([Wg2, Wg3], axis=1), ((0, 0), (0, 10)))

    cnt = _sc_degree_count(dst2d, ones_col, zeros1)
    po_emb, hs1a, hs1b, dinv = _tc_encoder(x, coords, cnt, W1, b1r, W2, b2r,
                                           Wg1, fvec, smask)
    s1a = _sc_segment_sum(hs1a, src2d, dst2d, zeros16, 16)
    s1b = _sc_segment_sum(hs1b, src2d, dst2d, zeros16, 16)
    hs2 = _tc_hidden(s1a, s1b, hs1a, hs1b, dinv, Wcat)
    s2 = _sc_segment_sum(hs2, src2d, dst2d, zeros16, 16)
    o6 = _tc_final(s2, hs2, dinv)
    return (o6[:, 0:3], o6[:, 3:6], po_emb)


# speedup vs baseline: 1.0558x; 1.0558x over previous
import functools

import numpy as np

import jax
import jax.numpy as jnp
from jax import lax
from jax.experimental import pallas as pl
from jax.experimental.pallas import tpu as pltpu
from jax.experimental.pallas import tpu_sc as plsc

_N = 50000
_E = 800000
_D_IN = 128
_ZDIM = 96
_ENC_DIM = 16
_PE_ALPHA = 0.1

_NC = 2
_NS = 16
_NW = _NC * _NS

_NP = 50048
_ZR = _NP // _NS

_EPAD = 819200
_EROWS = _EPAD // 128
_RB = 8
_NBLK = _EROWS // _RB
_T0 = 512

_BLK = 2000
_GRID = _N // _BLK


def _sc_mesh():
    return plsc.VectorSubcoreMesh(core_axis_name="c", subcore_axis_name="s")


def _worker_blocks(cid, sid):
    tc = jnp.where(cid == 0, _T0, _NBLK - _T0)
    base = jnp.where(cid == 0, 0, _T0)
    per = tc // _NS
    rem = tc - per * _NS
    start = base + sid * per + jnp.minimum(sid, rem)
    nblk = per + (sid < rem).astype(jnp.int32)
    return start, nblk


def _sc_degree_count(dst2d, ones_col, zeros_col):
    @functools.partial(
        pl.kernel,
        out_type=jax.ShapeDtypeStruct((_NC, _NP, 1), jnp.float32),
        mesh=_sc_mesh(),
        scratch_types=[
            pltpu.VMEM((_RB, 128), jnp.int32),
            pltpu.VMEM((128, 1), jnp.float32),
            pltpu.VMEM_SHARED((_NP, 1), jnp.float32),
            pltpu.SemaphoreType.DMA,
        ],
        compiler_params=pltpu.CompilerParams(use_tc_tiling_on_sc=False),
    )
    def k(dst_h, ones_h, zeros_h, out_h, idx_d, ones_v, acc, ssem):
        cid = lax.axis_index("c")
        sid = lax.axis_index("s")
        pltpu.sync_copy(zeros_h, acc.at[pl.ds(sid * _ZR, _ZR)])
        pltpu.sync_copy(ones_h, ones_v)
        plsc.subcore_barrier()
        start, nblk = _worker_blocks(cid, sid)

        def body(b, carry):
            pltpu.sync_copy(dst_h.at[pl.ds((start + b) * _RB, _RB)], idx_d)
            descs = [
                pltpu.async_copy(ones_v, acc.at[idx_d.at[j]], ssem, add=True)
                for j in range(_RB)
            ]
            for d in descs:
                d.wait()
            return carry

        lax.fori_loop(0, nblk, body, 0)
        plsc.subcore_barrier()
        pltpu.sync_copy(acc.at[pl.ds(sid * _ZR, _ZR)],
                        out_h.at[cid, pl.ds(sid * _ZR, _ZR)])

    return k(dst2d, ones_col, zeros_col)


def _sc_segment_sum(table, src2d, dst2d, zeros, width):
    @functools.partial(
        pl.kernel,
        out_type=jax.ShapeDtypeStruct((_NC, _NP, width), jnp.float32),
        mesh=_sc_mesh(),
        scratch_types=[
            pltpu.VMEM((_RB, 128), jnp.int32),
            pltpu.VMEM((_RB, 128), jnp.int32),
            pltpu.VMEM((_RB * 128, width), jnp.float32),
            pltpu.VMEM_SHARED((_NP, width), jnp.float32),
            pltpu.SemaphoreType.DMA,
            pltpu.SemaphoreType.DMA,
        ],
        compiler_params=pltpu.CompilerParams(use_tc_tiling_on_sc=False),
    )
    def k(table_h, src_h, dst_h, zeros_h, out_h,
          idx_s, idx_d, rows, acc, gsem, ssem):
        cid = lax.axis_index("c")
        sid = lax.axis_index("s")
        pltpu.sync_copy(zeros_h, acc.at[pl.ds(sid * _ZR, _ZR)])
        plsc.subcore_barrier()
        start, nblk = _worker_blocks(cid, sid)

        def body(b, carry):
            rbase = (start + b) * _RB
            pltpu.sync_copy(src_h.at[pl.ds(rbase, _RB)], idx_s)
            pltpu.sync_copy(dst_h.at[pl.ds(rbase, _RB)], idx_d)
            gd = [
                pltpu.async_copy(table_h.at[idx_s.at[j]],
                                 rows.at[pl.ds(j * 128, 128)], gsem)
                for j in range(_RB)
            ]
            for d in gd:
                d.wait()
            sd = [
                pltpu.async_copy(rows.at[pl.ds(j * 128, 128)],
                                 acc.at[idx_d.at[j]], ssem, add=True)
                for j in range(_RB)
            ]
            for d in sd:
                d.wait()
            return carry

        lax.fori_loop(0, nblk, body, 0)
        plsc.subcore_barrier()
        pltpu.sync_copy(acc.at[pl.ds(sid * _ZR, _ZR)],
                        out_h.at[cid, pl.ds(sid * _ZR, _ZR)])

    return k(table, src2d, dst2d, zeros)


def _tc_encoder(x, coords, cnt, W1, b1r, W2, b2r, Wg1, fvec, smask):
    def body(x_r, c_r, cnt0_r, cnt1_r, W1_r, b1_r, W2_r, b2_r, Wg1_r,
             fv_r, sm_r, po_r, hs1a_r, hs1b_r, di_r):
        xb = x_r[...]
        e0 = jnp.maximum(
            jnp.dot(xb, W1_r[...], preferred_element_type=jnp.float32)
            + b1_r[...], 0.0)
        fe = jnp.maximum(
            jnp.dot(e0, W2_r[...], preferred_element_type=jnp.float32)
            + b2_r[...], 0.0)
        g = jnp.dot(fe, Wg1_r[...], preferred_element_type=jnp.float32)
        deg = cnt0_r[0] + cnt1_r[0] + 1.0
        di = lax.rsqrt(deg)
        di_r[...] = di
        hs1 = g * di
        hs1a_r[...] = hs1[:, 0:16]
        hs1b_r[...] = hs1[:, 16:32]
        c = c_r[...]
        crep = jnp.concatenate(
            [jnp.broadcast_to(c[:, k:k + 1], (_BLK, 32)) for k in range(3)],
            axis=1)
        ang = crep * fv_r[...]
        sm = sm_r[...]
        po_r[...] = sm * jnp.sin(ang) + (1.0 - sm) * jnp.cos(ang) + fe

    return pl.pallas_call(
        body,
        grid=(_GRID,),
        in_specs=[
            pl.BlockSpec((_BLK, _D_IN), lambda i: (i, 0)),
            pl.BlockSpec((_BLK, 3), lambda i: (i, 0)),
            pl.BlockSpec((1, _BLK, 1), lambda i: (0, i, 0)),
            pl.BlockSpec((1, _BLK, 1), lambda i: (1, i, 0)),
            pl.BlockSpec((_D_IN, 512), lambda i: (0, 0)),
            pl.BlockSpec((1, 512), lambda i: (0, 0)),
            pl.BlockSpec((512, _ZDIM), lambda i: (0, 0)),
            pl.BlockSpec((1, _ZDIM), lambda i: (0, 0)),
            pl.BlockSpec((_ZDIM, 32), lambda i: (0, 0)),
            pl.BlockSpec((1, _ZDIM), lambda i: (0, 0)),
            pl.BlockSpec((1, _ZDIM), lambda i: (0, 0)),
        ],
        out_specs=[
            pl.BlockSpec((_BLK, _ZDIM), lambda i: (i, 0)),
            pl.BlockSpec((_BLK, 16), lambda i: (i, 0)),
            pl.BlockSpec((_BLK, 16), lambda i: (i, 0)),
            pl.BlockSpec((_BLK, 1), lambda i: (i, 0)),
        ],
        out_shape=[
            jax.ShapeDtypeStruct((_N, _ZDIM), jnp.float32),
            jax.ShapeDtypeStruct((_N, 16), jnp.float32),
            jax.ShapeDtypeStruct((_N, 16), jnp.float32),
            jax.ShapeDtypeStruct((_N, 1), jnp.float32),
        ],
    )(x, coords, cnt, cnt, W1, b1r, W2, b2r, Wg1, fvec, smask)


def _tc_hidden(s1a, s1b, hs1a, hs1b, dinv, Wcat):
    def body(s1a0_r, s1a1_r, s1b0_r, s1b1_r, hs1a_r, hs1b_r, di_r, Wc_r,
             hs2_r):
        di = di_r[...]
        ha = di * (s1a0_r[0] + s1a1_r[0] + hs1a_r[...])
        hb = di * (s1b0_r[0] + s1b1_r[0] + hs1b_r[...])
        h = jnp.tanh(jnp.concatenate([ha, hb], axis=1))
        hs2_r[...] = jnp.dot(h, Wc_r[...],
                             preferred_element_type=jnp.float32) * di

    half = pl.BlockSpec((_BLK, 16), lambda i: (i, 0))
    return pl.pallas_call(
        body,
        grid=(_GRID,),
        in_specs=[
            pl.BlockSpec((1, _BLK, 16), lambda i: (0, i, 0)),
            pl.BlockSpec((1, _BLK, 16), lambda i: (1, i, 0)),
            pl.BlockSpec((1, _BLK, 16), lambda i: (0, i, 0)),
            pl.BlockSpec((1, _BLK, 16), lambda i: (1, i, 0)),
            half,
            half,
            pl.BlockSpec((_BLK, 1), lambda i: (i, 0)),
            pl.BlockSpec((32, 16), lambda i: (0, 0)),
        ],
        out_specs=[pl.BlockSpec((_BLK, 16), lambda i: (i, 0))],
        out_shape=[jax.ShapeDtypeStruct((_N, 16), jnp.float32)],
    )(s1a, s1a, s1b, s1b, hs1a, hs1b, dinv, Wcat)[0]


def _tc_final(s2, hs2, dinv):
    def body(s20_r, s21_r, hs2_r, di_r, o_r):
        o_r[...] = di_r[...] * (s20_r[0] + s21_r[0] + hs2_r[...])

    return pl.pallas_call(
        body,
        grid=(_GRID,),
        in_specs=[
            pl.BlockSpec((1, _BLK, 16), lambda i: (0, i, 0)),
            pl.BlockSpec((1, _BLK, 16), lambda i: (1, i, 0)),
            pl.BlockSpec((_BLK, 16), lambda i: (i, 0)),
            pl.BlockSpec((_BLK, 1), lambda i: (i, 0)),
        ],
        out_specs=[pl.BlockSpec((_BLK, 16), lambda i: (i, 0))],
        out_shape=[jax.ShapeDtypeStruct((_N, 16), jnp.float32)],
    )(s2, s2, hs2, dinv)[0]


def kernel(x, edge_index, coords, W1, b1, W2, b2, Wg1, Wg2, Wg3):
    pad = _EPAD - _E
    src2d = jnp.concatenate(
        [edge_index[0], jnp.zeros((pad,), jnp.int32)]).reshape(_EROWS, 128)
    dst2d = jnp.concatenate(
        [edge_index[1], jnp.full((pad,), _N, jnp.int32)]).reshape(_EROWS, 128)

    zeros16 = jnp.zeros((_ZR, 16), jnp.float32)
    zeros1 = jnp.zeros((_ZR, 1), jnp.float32)
    ones_col = jnp.ones((128, 1), jnp.float32)

    kk = np.arange(_ZDIM) % 32
    fvec = jnp.asarray(
        (_PE_ALPHA * (2.0 ** (kk % _ENC_DIM))).reshape(1, _ZDIM), jnp.float32)
    smask = jnp.asarray((kk < _ENC_DIM).astype(np.float32).reshape(1, _ZDIM))

    b1r = b1.reshape(1, 512)
    b2r = b2.reshape(1, _ZDIM)
    Wcat = jnp.pad(jnp.concatenate([Wg2, Wg3], axis=1), ((0, 0), (0, 10)))

    cnt = _sc_degree_count(dst2d, ones_col, zeros1)
    po_emb, hs1a, hs1b, dinv = _tc_encoder(x, coords, cnt, W1, b1r, W2, b2r,
                                           Wg1, fvec, smask)
    s1a = _sc_segment_sum(hs1a, src2d, dst2d, zeros16, 16)
    s1b = _sc_segment_sum(hs1b, src2d, dst2d, zeros16, 16)
    hs2 = _tc_hidden(s1a, s1b, hs1a, hs1b, dinv, Wcat)
    s2 = _sc_segment_sum(hs2, src2d, dst2d, zeros16, 16)
    o6 = _tc_final(s2, hs2, dinv)
    return (o6[:, 0:3], o6[:, 3:6], po_emb)


# speedup vs baseline: 1.0975x; 1.0395x over previous
import functools

import numpy as np

import jax
import jax.numpy as jnp
from jax import lax
from jax.experimental import pallas as pl
from jax.experimental.pallas import tpu as pltpu
from jax.experimental.pallas import tpu_sc as plsc

_N = 50000
_E = 800000
_D_IN = 128
_ZDIM = 96
_ENC_DIM = 16
_PE_ALPHA = 0.1

_NC = 2
_NS = 16
_NW = _NC * _NS

_NP = 50048
_ZR = _NP // _NS

_EPAD = 819200
_EROWS = _EPAD // 128
_RB = 8
_NBLK = _EROWS // _RB
_T0 = 512

_BLK = 2000
_GRID = _N // _BLK


def _sc_mesh():
    return plsc.VectorSubcoreMesh(core_axis_name="c", subcore_axis_name="s")


def _worker_blocks(cid, sid):
    tc = jnp.where(cid == 0, _T0, _NBLK - _T0)
    base = jnp.where(cid == 0, 0, _T0)
    per = tc // _NS
    rem = tc - per * _NS
    start = base + sid * per + jnp.minimum(sid, rem)
    nblk = per + (sid < rem).astype(jnp.int32)
    return start, nblk


def _sc_degree_count(dst2d, ones_col, zeros_col):
    @functools.partial(
        pl.kernel,
        out_type=jax.ShapeDtypeStruct((_NC, _NP, 1), jnp.float32),
        mesh=_sc_mesh(),
        scratch_types=[
            pltpu.VMEM((_RB, 128), jnp.int32),
            pltpu.VMEM((128, 1), jnp.float32),
            pltpu.VMEM_SHARED((_NP, 1), jnp.float32),
            pltpu.SemaphoreType.DMA,
        ],
        compiler_params=pltpu.CompilerParams(use_tc_tiling_on_sc=False),
    )
    def k(dst_h, ones_h, zeros_h, out_h, idx_d, ones_v, acc, ssem):
        cid = lax.axis_index("c")
        sid = lax.axis_index("s")
        pltpu.sync_copy(zeros_h, acc.at[pl.ds(sid * _ZR, _ZR)])
        pltpu.sync_copy(ones_h, ones_v)
        plsc.subcore_barrier()
        start, nblk = _worker_blocks(cid, sid)

        def body(b, carry):
            pltpu.sync_copy(dst_h.at[pl.ds((start + b) * _RB, _RB)], idx_d)
            descs = [
                pltpu.async_copy(ones_v, acc.at[idx_d.at[j]], ssem, add=True)
                for j in range(_RB)
            ]
            for d in descs:
                d.wait()
            return carry

        lax.fori_loop(0, nblk, body, 0)
        plsc.subcore_barrier()
        pltpu.sync_copy(acc.at[pl.ds(sid * _ZR, _ZR)],
                        out_h.at[cid, pl.ds(sid * _ZR, _ZR)])

    return k(dst2d, ones_col, zeros_col)


def _sc_segment_sum(table, src2d, dst2d, zeros, width):
    @functools.partial(
        pl.kernel,
        out_type=jax.ShapeDtypeStruct((_NC, _NP, width), jnp.float32),
        mesh=_sc_mesh(),
        scratch_types=[
            pltpu.VMEM((_RB, 128), jnp.int32),
            pltpu.VMEM((_RB, 128), jnp.int32),
            pltpu.VMEM((_RB * 128, width), jnp.float32),
            pltpu.VMEM((_RB * 128, width), jnp.float32),
            pltpu.VMEM_SHARED((_NP, width), jnp.float32),
            pltpu.SemaphoreType.DMA,
            pltpu.SemaphoreType.DMA,
            pltpu.SemaphoreType.DMA,
            pltpu.SemaphoreType.DMA,
        ],
        compiler_params=pltpu.CompilerParams(use_tc_tiling_on_sc=False),
    )
    def k(table_h, src_h, dst_h, zeros_h, out_h,
          idx_s, idx_d, rows0, rows1, acc, gsem0, gsem1, ssem0, ssem1):
        cid = lax.axis_index("c")
        sid = lax.axis_index("s")
        pltpu.sync_copy(zeros_h, acc.at[pl.ds(sid * _ZR, _ZR)])
        plsc.subcore_barrier()
        start, nblk = _worker_blocks(cid, sid)
        half = _RB // 2

        def issue_gathers(rbase, idx, rows, sem):
            pltpu.sync_copy(src_h.at[pl.ds(rbase, _RB)], idx)
            return [
                pltpu.async_copy(table_h.at[idx.at[j]],
                                 rows.at[pl.ds(j * 128, 128)], sem)
                for j in range(_RB)
            ]

        def issue_scatters(rbase, idx, rows, sem):
            pltpu.sync_copy(dst_h.at[pl.ds(rbase, _RB)], idx)
            return [
                pltpu.async_copy(rows.at[pl.ds(j * 128, 128)],
                                 acc.at[idx.at[j]], sem, add=True)
                for j in range(_RB)
            ]

        def body(b, carry):
            rb0 = (start + 2 * b) * _RB
            rb1 = rb0 + _RB
            g0 = issue_gathers(rb0, idx_s, rows0, gsem0)
            g1 = issue_gathers(rb1, idx_d, rows1, gsem1)
            for d in g0:
                d.wait()
            s0 = issue_scatters(rb0, idx_s, rows0, ssem0)
            for d in g1:
                d.wait()
            s1 = issue_scatters(rb1, idx_d, rows1, ssem1)
            for d in s0:
                d.wait()
            for d in s1:
                d.wait()
            return carry

        lax.fori_loop(0, nblk // 2, body, 0)
        plsc.subcore_barrier()
        pltpu.sync_copy(acc.at[pl.ds(sid * _ZR, _ZR)],
                        out_h.at[cid, pl.ds(sid * _ZR, _ZR)])

    return k(table, src2d, dst2d, zeros)


def _tc_encoder(x, coords, cnt, W1, b1r, W2, b2r, Wg1, fvec, smask):
    def body(x_r, c_r, cnt0_r, cnt1_r, W1_r, b1_r, W2_r, b2_r, Wg1_r,
             fv_r, sm_r, po_r, hs1a_r, hs1b_r, di_r):
        xb = x_r[...]
        e0 = jnp.maximum(
            jnp.dot(xb, W1_r[...], preferred_element_type=jnp.float32)
            + b1_r[...], 0.0)
        fe = jnp.maximum(
            jnp.dot(e0, W2_r[...], preferred_element_type=jnp.float32)
            + b2_r[...], 0.0)
        g = jnp.dot(fe, Wg1_r[...], preferred_element_type=jnp.float32)
        deg = cnt0_r[0] + cnt1_r[0] + 1.0
        di = lax.rsqrt(deg)
        di_r[...] = di
        hs1 = g * di
        hs1a_r[...] = hs1[:, 0:16]
        hs1b_r[...] = hs1[:, 16:32]
        c = c_r[...]
        crep = jnp.concatenate(
            [jnp.broadcast_to(c[:, k:k + 1], (_BLK, 32)) for k in range(3)],
            axis=1)
        ang = crep * fv_r[...]
        sm = sm_r[...]
        po_r[...] = sm * jnp.sin(ang) + (1.0 - sm) * jnp.cos(ang) + fe

    return pl.pallas_call(
        body,
        grid=(_GRID,),
        in_specs=[
            pl.BlockSpec((_BLK, _D_IN), lambda i: (i, 0)),
            pl.BlockSpec((_BLK, 3), lambda i: (i, 0)),
            pl.BlockSpec((1, _BLK, 1), lambda i: (0, i, 0)),
            pl.BlockSpec((1, _BLK, 1), lambda i: (1, i, 0)),
            pl.BlockSpec((_D_IN, 512), lambda i: (0, 0)),
            pl.BlockSpec((1, 512), lambda i: (0, 0)),
            pl.BlockSpec((512, _ZDIM), lambda i: (0, 0)),
            pl.BlockSpec((1, _ZDIM), lambda i: (0, 0)),
            pl.BlockSpec((_ZDIM, 32), lambda i: (0, 0)),
            pl.BlockSpec((1, _ZDIM), lambda i: (0, 0)),
            pl.BlockSpec((1, _ZDIM), lambda i: (0, 0)),
        ],
        out_specs=[
            pl.BlockSpec((_BLK, _ZDIM), lambda i: (i, 0)),
            pl.BlockSpec((_BLK, 16), lambda i: (i, 0)),
            pl.BlockSpec((_BLK, 16), lambda i: (i, 0)),
            pl.BlockSpec((_BLK, 1), lambda i: (i, 0)),
        ],
        out_shape=[
            jax.ShapeDtypeStruct((_N, _ZDIM), jnp.float32),
            jax.ShapeDtypeStruct((_N, 16), jnp.float32),
            jax.ShapeDtypeStruct((_N, 16), jnp.float32),
            jax.ShapeDtypeStruct((_N, 1), jnp.float32),
        ],
    )(x, coords, cnt, cnt, W1, b1r, W2, b2r, Wg1, fvec, smask)


def _tc_hidden(s1a, s1b, hs1a, hs1b, dinv, Wcat):
    def body(s1a0_r, s1a1_r, s1b0_r, s1b1_r, hs1a_r, hs1b_r, di_r, Wc_r,
             hs2_r):
        di = di_r[...]
        ha = di * (s1a0_r[0] + s1a1_r[0] + hs1a_r[...])
        hb = di * (s1b0_r[0] + s1b1_r[0] + hs1b_r[...])
        h = jnp.tanh(jnp.concatenate([ha, hb], axis=1))
        hs2_r[...] = jnp.dot(h, Wc_r[...],
                             preferred_element_type=jnp.float32) * di

    half = pl.BlockSpec((_BLK, 16), lambda i: (i, 0))
    return pl.pallas_call(
        body,
        grid=(_GRID,),
        in_specs=[
            pl.BlockSpec((1, _BLK, 16), lambda i: (0, i, 0)),
            pl.BlockSpec((1, _BLK, 16), lambda i: (1, i, 0)),
            pl.BlockSpec((1, _BLK, 16), lambda i: (0, i, 0)),
            pl.BlockSpec((1, _BLK, 16), lambda i: (1, i, 0)),
            half,
            half,
            pl.BlockSpec((_BLK, 1), lambda i: (i, 0)),
            pl.BlockSpec((32, 16), lambda i: (0, 0)),
        ],
        out_specs=[pl.BlockSpec((_BLK, 16), lambda i: (i, 0))],
        out_shape=[jax.ShapeDtypeStruct((_N, 16), jnp.float32)],
    )(s1a, s1a, s1b, s1b, hs1a, hs1b, dinv, Wcat)[0]


def _tc_final(s2, hs2, dinv):
    def body(s20_r, s21_r, hs2_r, di_r, o_r):
        o_r[...] = di_r[...] * (s20_r[0] + s21_r[0] + hs2_r[...])

    return pl.pallas_call(
        body,
        grid=(_GRID,),
        in_specs=[
            pl.BlockSpec((1, _BLK, 16), lambda i: (0, i, 0)),
            pl.BlockSpec((1, _BLK, 16), lambda i: (1, i, 0)),
            pl.BlockSpec((_BLK, 16), lambda i: (i, 0)),
            pl.BlockSpec((_BLK, 1), lambda i: (i, 0)),
        ],
        out_specs=[pl.BlockSpec((_BLK, 16), lambda i: (i, 0))],
        out_shape=[jax.ShapeDtypeStruct((_N, 16), jnp.float32)],
    )(s2, s2, hs2, dinv)[0]


def kernel(x, edge_index, coords, W1, b1, W2, b2, Wg1, Wg2, Wg3):
    pad = _EPAD - _E
    src2d = jnp.concatenate(
        [edge_index[0], jnp.zeros((pad,), jnp.int32)]).reshape(_EROWS, 128)
    dst2d = jnp.concatenate(
        [edge_index[1], jnp.full((pad,), _N, jnp.int32)]).reshape(_EROWS, 128)

    zeros16 = jnp.zeros((_ZR, 16), jnp.float32)
    zeros1 = jnp.zeros((_ZR, 1), jnp.float32)
    ones_col = jnp.ones((128, 1), jnp.float32)

    kk = np.arange(_ZDIM) % 32
    fvec = jnp.asarray(
        (_PE_ALPHA * (2.0 ** (kk % _ENC_DIM))).reshape(1, _ZDIM), jnp.float32)
    smask = jnp.asarray((kk < _ENC_DIM).astype(np.float32).reshape(1, _ZDIM))

    b1r = b1.reshape(1, 512)
    b2r = b2.reshape(1, _ZDIM)
    Wcat = jnp.pad(jnp.concatenate([Wg2, Wg3], axis=1), ((0, 0), (0, 10)))

    cnt = _sc_degree_count(dst2d, ones_col, zeros1)
    po_emb, hs1a, hs1b, dinv = _tc_encoder(x, coords, cnt, W1, b1r, W2, b2r,
                                           Wg1, fvec, smask)
    s1a = _sc_segment_sum(hs1a, src2d, dst2d, zeros16, 16)
    s1b = _sc_segment_sum(hs1b, src2d, dst2d, zeros16, 16)
    hs2 = _tc_hidden(s1a, s1b, hs1a, hs1b, dinv, Wcat)
    s2 = _sc_segment_sum(hs2, src2d, dst2d, zeros16, 16)
    o6 = _tc_final(s2, hs2, dinv)
    return (o6[:, 0:3], o6[:, 3:6], po_emb)
